# Initial kernel scaffold; baseline (speedup 1.0000x reference)
import jax
import jax.numpy as jnp
from jax.experimental import pallas as pl


def kernel(x, edge_index, batch, W0, W1, W2, W3, W4, lin1_W, lin1_b, lin2_W, lin2_b):
    raise NotImplementedError("write your pallas kernel here")


# speedup vs baseline: 73.7868x; 73.7868x over previous
import functools

import jax
import jax.numpy as jnp
from jax import lax
from jax.experimental import pallas as pl
from jax.experimental.pallas import tpu as pltpu
from jax.experimental.pallas import tpu_sc as plsc

N = 10000
NPAD = 10240
NROWS = 80
E = 160000
NTILES = 16
EPT = E // NTILES
EPT_PAD = NPAD
ECHUNKS = EPT // 16
ALPHA = 0.1
NITER = 10

BN = 1024
NB = NPAD // BN


def _sc_body(esrc, edst, w_out, src_v, dst_v, z_v, u_v, dinv_v, agg_v, w_v,
             zero_v, iota_v, shared):
    c = lax.axis_index("c")
    s = lax.axis_index("s")

    pltpu.sync_copy(esrc.at[s], src_v)
    pltpu.sync_copy(edst.at[s], dst_v)

    ones16 = jnp.ones((16,), jnp.float32)
    zeros16 = jnp.zeros((16,), jnp.float32)

    def init_row(r, carry):
        for j in range(8):
            sl = pl.ds(j * 16, 16)
            zero_v[r, sl] = zeros16
            agg_v[r, sl] = zeros16
            u_v[r, sl] = ones16
        return carry

    lax.fori_loop(0, NROWS, init_row, 0)

    for t in range((NPAD - N) // 16):
        flat = N + t * 16
        u_v[flat >> 7, pl.ds(flat & 127, 16)] = zeros16

    i16 = lax.iota(jnp.int32, 16)
    for j in range(NROWS // 16):
        iota_v[pl.ds(j * 16, 16)] = i16 + j * 16

    @pl.when(s == 0)
    def _():
        pltpu.sync_copy(zero_v, shared)

    plsc.subcore_barrier()

    def deg_step(i, carry):
        r = i >> 3
        co = (i & 7) * 16
        sv = src_v[r, pl.ds(co, 16)]
        plsc.addupdate_scatter(agg_v, [sv >> 7, sv & 127], ones16)
        return carry

    lax.fori_loop(0, ECHUNKS, deg_step, 0)

    pltpu.sync_copy(agg_v, shared.at[iota_v], add=True)
    plsc.subcore_barrier()
    pltpu.sync_copy(shared, dinv_v)
    plsc.subcore_barrier()

    @pl.when(s == 0)
    def _():
        pltpu.sync_copy(zero_v, shared)

    def dinv_row(r, carry):
        for j in range(8):
            sl = pl.ds(j * 16, 16)
            xdeg = dinv_v[r, sl] + 1.0
            bi = 0x5F3759DF - lax.shift_right_logical(plsc.bitcast(xdeg, jnp.int32), 1)
            y = plsc.bitcast(bi, jnp.float32)
            y = y * (1.5 - 0.5 * xdeg * y * y)
            y = y * (1.5 - 0.5 * xdeg * y * y)
            y = y * (1.5 - 0.5 * xdeg * y * y)
            dinv_v[r, sl] = y
            agg_v[r, sl] = zeros16
            w_v[r, sl] = 0.1 * u_v[r, sl]
        return carry

    lax.fori_loop(0, NROWS, dinv_row, 0)
    plsc.subcore_barrier()

    def iteration(k, carry):
        coef = jnp.where(k < NITER, jnp.float32(ALPHA), jnp.float32(1.0))

        def z_row(r, cc):
            for j in range(8):
                sl = pl.ds(j * 16, 16)
                z_v[r, sl] = dinv_v[r, sl] * u_v[r, sl]
            return cc

        lax.fori_loop(0, NROWS, z_row, 0)

        def edge_step(i, cc):
            r = i >> 3
            co = (i & 7) * 16
            sv = src_v[r, pl.ds(co, 16)]
            dv = dst_v[r, pl.ds(co, 16)]
            g = plsc.load_gather(z_v, [sv >> 7, sv & 127])
            plsc.addupdate_scatter(agg_v, [dv >> 7, dv & 127], g)
            return cc

        lax.fori_loop(0, ECHUNKS, edge_step, 0)

        pltpu.sync_copy(agg_v, shared.at[iota_v], add=True)
        plsc.subcore_barrier()
        pltpu.sync_copy(shared, agg_v)
        plsc.subcore_barrier()

        @pl.when(s == 0)
        def _():
            pltpu.sync_copy(zero_v, shared)

        def upd_row(r, cc):
            for j in range(8):
                sl = pl.ds(j * 16, 16)
                un = (1.0 - ALPHA) * dinv_v[r, sl] * (agg_v[r, sl] + z_v[r, sl])
                u_v[r, sl] = un
                w_v[r, sl] = w_v[r, sl] + coef * un
                agg_v[r, sl] = zeros16
            return cc

        lax.fori_loop(0, NROWS, upd_row, 0)
        plsc.subcore_barrier()
        return carry

    lax.fori_loop(1, NITER + 1, iteration, 0)

    @pl.when(jnp.logical_and(c == 0, s == 0))
    def _():
        pltpu.sync_copy(w_v, w_out)


_sc_ppr = pl.kernel(
    _sc_body,
    out_type=jax.ShapeDtypeStruct((NROWS, 128), jnp.float32),
    mesh=plsc.VectorSubcoreMesh(core_axis_name="c", subcore_axis_name="s"),
    compiler_params=pltpu.CompilerParams(needs_layout_passes=False),
    scratch_types=[
        pltpu.VMEM((NROWS, 128), jnp.int32),
        pltpu.VMEM((NROWS, 128), jnp.int32),
        pltpu.VMEM((NROWS, 128), jnp.float32),
        pltpu.VMEM((NROWS, 128), jnp.float32),
        pltpu.VMEM((NROWS, 128), jnp.float32),
        pltpu.VMEM((NROWS, 128), jnp.float32),
        pltpu.VMEM((NROWS, 128), jnp.float32),
        pltpu.VMEM((NROWS, 128), jnp.float32),
        pltpu.VMEM((NROWS,), jnp.int32),
        pltpu.VMEM_SHARED((NROWS, 128), jnp.float32),
    ],
)


def _tc_body(x_ref, w_ref, w0_ref, w1_ref, w2_ref, w3_ref, w4_ref,
             l1w_ref, l1b_ref, l2w_ref, l2b_ref, out_ref, acc_ref):
    i = pl.program_id(0)
    h = jnp.maximum(jnp.dot(x_ref[...], w0_ref[...], preferred_element_type=jnp.float32, precision=jax.lax.Precision.HIGHEST), 0.0)
    h = jnp.maximum(jnp.dot(h, w1_ref[...], preferred_element_type=jnp.float32, precision=jax.lax.Precision.HIGHEST), 0.0)
    h = jnp.maximum(jnp.dot(h, w2_ref[...], preferred_element_type=jnp.float32, precision=jax.lax.Precision.HIGHEST), 0.0)
    h = jnp.maximum(jnp.dot(h, w3_ref[...], preferred_element_type=jnp.float32, precision=jax.lax.Precision.HIGHEST), 0.0)
    contrib = jnp.dot(w_ref[0], h, preferred_element_type=jnp.float32, precision=jax.lax.Precision.HIGHEST)

    @pl.when(i == 0)
    def _():
        acc_ref[...] = jnp.zeros_like(acc_ref)

    acc_ref[...] += contrib

    @pl.when(i == NB - 1)
    def _():
        pooled = jnp.dot(acc_ref[...], w4_ref[...], preferred_element_type=jnp.float32, precision=jax.lax.Precision.HIGHEST)
        h2 = jnp.maximum(
            jnp.dot(pooled, l1w_ref[...], preferred_element_type=jnp.float32, precision=jax.lax.Precision.HIGHEST) + l1b_ref[...],
            0.0)
        out_ref[...] = jnp.dot(h2, l2w_ref[...], preferred_element_type=jnp.float32, precision=jax.lax.Precision.HIGHEST) + l2b_ref[...]


def _rep(shape):
    return pl.BlockSpec(shape, lambda i: tuple(0 for _ in shape))


_tc_mlp = pl.pallas_call(
    _tc_body,
    grid=(NB,),
    in_specs=[
        pl.BlockSpec((BN, 256), lambda i: (i, 0)),
        pl.BlockSpec((1, 1, BN), lambda i: (i, 0, 0)),
        _rep((256, 512)),
        _rep((512, 512)),
        _rep((512, 512)),
        _rep((512, 512)),
        _rep((512, 512)),
        _rep((512, 512)),
        _rep((1, 512)),
        _rep((512, 128)),
        _rep((1, 128)),
    ],
    out_specs=pl.BlockSpec((1, 128), lambda i: (0, 0)),
    out_shape=jax.ShapeDtypeStruct((1, 128), jnp.float32),
    scratch_shapes=[pltpu.VMEM((1, 512), jnp.float32)],
)


def kernel(x, edge_index, batch, W0, W1, W2, W3, W4, lin1_W, lin1_b, lin2_W, lin2_b):
    src = edge_index[0].reshape(NTILES, EPT)
    dst = edge_index[1].reshape(NTILES, EPT)
    pad = ((0, 0), (0, EPT_PAD - EPT))
    esrc3 = jnp.pad(src, pad).reshape(NTILES, NROWS, 128)
    edst3 = jnp.pad(dst, pad).reshape(NTILES, NROWS, 128)
    w2d = _sc_ppr(esrc3, edst3)
    w3 = w2d.reshape(NB, 1, BN)

    x_pad = jnp.pad(x, ((0, NPAD - N), (0, 0)))
    out = _tc_mlp(x_pad, w3, W0, W1, W2, W3, W4,
                  lin1_W, lin1_b.reshape(1, 512), lin2_W, lin2_b.reshape(1, 128))
    return out


# speedup vs baseline: 74.4983x; 1.0096x over previous
import functools

import jax
import jax.numpy as jnp
from jax import lax
from jax.experimental import pallas as pl
from jax.experimental.pallas import tpu as pltpu
from jax.experimental.pallas import tpu_sc as plsc

N = 10000
NPAD = 10240
NROWS = 80
E = 160000
NTILES = 16
EPT = E // NTILES
EPT_PAD = NPAD
ECHUNKS = EPT // 16
ALPHA = 0.1
NITER = 10

BN = 1024
NB = NPAD // BN


def _sc_body(esrc, edst, w_out, src_v, dst_v, z_v, u_v, dinv_v, agg_v, w_v,
             zero_v, iota_v, shared):
    c = lax.axis_index("c")
    s = lax.axis_index("s")

    pltpu.sync_copy(esrc.at[s], src_v)
    pltpu.sync_copy(edst.at[s], dst_v)

    ones16 = jnp.ones((16,), jnp.float32)
    zeros16 = jnp.zeros((16,), jnp.float32)

    def init_row(r, carry):
        for j in range(8):
            sl = pl.ds(j * 16, 16)
            zero_v[r, sl] = zeros16
            agg_v[r, sl] = zeros16
            u_v[r, sl] = ones16
        return carry

    lax.fori_loop(0, NROWS, init_row, 0)

    for t in range((NPAD - N) // 16):
        flat = N + t * 16
        u_v[flat >> 7, pl.ds(flat & 127, 16)] = zeros16

    i16 = lax.iota(jnp.int32, 16)
    for j in range(NROWS // 16):
        iota_v[pl.ds(j * 16, 16)] = i16 + j * 16

    @pl.when(s == 0)
    def _():
        pltpu.sync_copy(zero_v, shared)

    plsc.subcore_barrier()

    def deg_step(i, carry):
        r = i >> 3
        co = (i & 7) * 16
        sv = src_v[r, pl.ds(co, 16)]
        plsc.addupdate_scatter(agg_v, [sv >> 7, sv & 127], ones16)
        return carry

    lax.fori_loop(0, ECHUNKS, deg_step, 0)

    pltpu.sync_copy(agg_v, shared.at[iota_v], add=True)
    plsc.subcore_barrier()
    pltpu.sync_copy(shared, dinv_v)
    plsc.subcore_barrier()

    @pl.when(s == 0)
    def _():
        pltpu.sync_copy(zero_v, shared)

    def dinv_row(r, carry):
        for j in range(8):
            sl = pl.ds(j * 16, 16)
            xdeg = dinv_v[r, sl] + 1.0
            bi = 0x5F3759DF - lax.shift_right_logical(plsc.bitcast(xdeg, jnp.int32), 1)
            y = plsc.bitcast(bi, jnp.float32)
            y = y * (1.5 - 0.5 * xdeg * y * y)
            y = y * (1.5 - 0.5 * xdeg * y * y)
            y = y * (1.5 - 0.5 * xdeg * y * y)
            dinv_v[r, sl] = y
            agg_v[r, sl] = zeros16
            w_v[r, sl] = 0.1 * u_v[r, sl]
        return carry

    lax.fori_loop(0, NROWS, dinv_row, 0)
    plsc.subcore_barrier()

    def iteration(k, carry):
        coef = jnp.where(k < NITER, jnp.float32(ALPHA), jnp.float32(1.0))

        def z_row(r, cc):
            for j in range(8):
                sl = pl.ds(j * 16, 16)
                z_v[r, sl] = dinv_v[r, sl] * u_v[r, sl]
            return cc

        lax.fori_loop(0, NROWS, z_row, 0)

        def edge_step(i, cc):
            r = i >> 3
            co = (i & 7) * 16
            sv = src_v[r, pl.ds(co, 16)]
            dv = dst_v[r, pl.ds(co, 16)]
            g = plsc.load_gather(z_v, [sv >> 7, sv & 127])
            plsc.addupdate_scatter(agg_v, [dv >> 7, dv & 127], g)
            return cc

        lax.fori_loop(0, ECHUNKS, edge_step, 0)

        pltpu.sync_copy(agg_v, shared.at[iota_v], add=True)
        plsc.subcore_barrier()
        pltpu.sync_copy(shared, agg_v)
        plsc.subcore_barrier()

        @pl.when(s == 0)
        def _():
            pltpu.sync_copy(zero_v, shared)

        def upd_row(r, cc):
            for j in range(8):
                sl = pl.ds(j * 16, 16)
                un = (1.0 - ALPHA) * dinv_v[r, sl] * (agg_v[r, sl] + z_v[r, sl])
                u_v[r, sl] = un
                w_v[r, sl] = w_v[r, sl] + coef * un
                agg_v[r, sl] = zeros16
            return cc

        lax.fori_loop(0, NROWS, upd_row, 0)
        plsc.subcore_barrier()
        return carry

    lax.fori_loop(1, NITER + 1, iteration, 0)

    @pl.when(jnp.logical_and(c == 0, s == 0))
    def _():
        pltpu.sync_copy(w_v, w_out)


_sc_ppr = pl.kernel(
    _sc_body,
    out_type=jax.ShapeDtypeStruct((NROWS, 128), jnp.float32),
    mesh=plsc.VectorSubcoreMesh(core_axis_name="c", subcore_axis_name="s",
                                num_cores=1),
    compiler_params=pltpu.CompilerParams(needs_layout_passes=False),
    scratch_types=[
        pltpu.VMEM((NROWS, 128), jnp.int32),
        pltpu.VMEM((NROWS, 128), jnp.int32),
        pltpu.VMEM((NROWS, 128), jnp.float32),
        pltpu.VMEM((NROWS, 128), jnp.float32),
        pltpu.VMEM((NROWS, 128), jnp.float32),
        pltpu.VMEM((NROWS, 128), jnp.float32),
        pltpu.VMEM((NROWS, 128), jnp.float32),
        pltpu.VMEM((NROWS, 128), jnp.float32),
        pltpu.VMEM((NROWS,), jnp.int32),
        pltpu.VMEM_SHARED((NROWS, 128), jnp.float32),
    ],
)


def _tc_body(x_ref, w_ref, w0_ref, w1_ref, w2_ref, w3_ref, w4_ref,
             l1w_ref, l1b_ref, l2w_ref, l2b_ref, out_ref, acc_ref):
    i = pl.program_id(0)
    h = jnp.maximum(jnp.dot(x_ref[...], w0_ref[...], preferred_element_type=jnp.float32, precision=jax.lax.Precision.HIGHEST), 0.0)
    h = jnp.maximum(jnp.dot(h, w1_ref[...], preferred_element_type=jnp.float32, precision=jax.lax.Precision.HIGHEST), 0.0)
    h = jnp.maximum(jnp.dot(h, w2_ref[...], preferred_element_type=jnp.float32, precision=jax.lax.Precision.HIGHEST), 0.0)
    h = jnp.maximum(jnp.dot(h, w3_ref[...], preferred_element_type=jnp.float32, precision=jax.lax.Precision.HIGHEST), 0.0)
    contrib = jnp.dot(w_ref[0], h, preferred_element_type=jnp.float32, precision=jax.lax.Precision.HIGHEST)

    @pl.when(i == 0)
    def _():
        acc_ref[...] = jnp.zeros_like(acc_ref)

    acc_ref[...] += contrib

    @pl.when(i == NB - 1)
    def _():
        pooled = jnp.dot(acc_ref[...], w4_ref[...], preferred_element_type=jnp.float32, precision=jax.lax.Precision.HIGHEST)
        h2 = jnp.maximum(
            jnp.dot(pooled, l1w_ref[...], preferred_element_type=jnp.float32, precision=jax.lax.Precision.HIGHEST) + l1b_ref[...],
            0.0)
        out_ref[...] = jnp.dot(h2, l2w_ref[...], preferred_element_type=jnp.float32, precision=jax.lax.Precision.HIGHEST) + l2b_ref[...]


def _rep(shape):
    return pl.BlockSpec(shape, lambda i: tuple(0 for _ in shape))


_tc_mlp = pl.pallas_call(
    _tc_body,
    grid=(NB,),
    in_specs=[
        pl.BlockSpec((BN, 256), lambda i: (i, 0)),
        pl.BlockSpec((1, 1, BN), lambda i: (i, 0, 0)),
        _rep((256, 512)),
        _rep((512, 512)),
        _rep((512, 512)),
        _rep((512, 512)),
        _rep((512, 512)),
        _rep((512, 512)),
        _rep((1, 512)),
        _rep((512, 128)),
        _rep((1, 128)),
    ],
    out_specs=pl.BlockSpec((1, 128), lambda i: (0, 0)),
    out_shape=jax.ShapeDtypeStruct((1, 128), jnp.float32),
    scratch_shapes=[pltpu.VMEM((1, 512), jnp.float32)],
)


def kernel(x, edge_index, batch, W0, W1, W2, W3, W4, lin1_W, lin1_b, lin2_W, lin2_b):
    src = edge_index[0].reshape(NTILES, EPT)
    dst = edge_index[1].reshape(NTILES, EPT)
    pad = ((0, 0), (0, EPT_PAD - EPT))
    esrc3 = jnp.pad(src, pad).reshape(NTILES, NROWS, 128)
    edst3 = jnp.pad(dst, pad).reshape(NTILES, NROWS, 128)
    w2d = _sc_ppr(esrc3, edst3)
    w3 = w2d.reshape(NB, 1, BN)

    x_pad = jnp.pad(x, ((0, NPAD - N), (0, 0)))
    out = _tc_mlp(x_pad, w3, W0, W1, W2, W3, W4,
                  lin1_W, lin1_b.reshape(1, 512), lin2_W, lin2_b.reshape(1, 128))
    return out


# speedup vs baseline: 122.6398x; 1.6462x over previous
import functools

import jax
import jax.numpy as jnp
from jax import lax
from jax.experimental import pallas as pl
from jax.experimental.pallas import tpu as pltpu
from jax.experimental.pallas import tpu_sc as plsc

N = 10000
NPAD = 10240
NROWS = 80
E = 160000
NTILES = 16
EPT = E // NTILES
EPT_PAD = NPAD
ECHUNKS = EPT // 16
ALPHA = 0.1
NITER = 10

BN = 1024
NB = NPAD // BN


def _sc_body(esrc, edst, w_out, src_v, dst_v, z_v, u_v, dinv_v, agg_v, w_v,
             zero_v, iota_v, shared):
    c = lax.axis_index("c")
    s = lax.axis_index("s")

    pltpu.sync_copy(esrc.at[s], src_v)
    pltpu.sync_copy(edst.at[s], dst_v)

    ones16 = jnp.ones((16,), jnp.float32)
    zeros16 = jnp.zeros((16,), jnp.float32)

    def init_row(r, carry):
        for j in range(8):
            sl = pl.ds(j * 16, 16)
            zero_v[r, sl] = zeros16
            agg_v[r, sl] = zeros16
            u_v[r, sl] = ones16
        return carry

    lax.fori_loop(0, NROWS, init_row, 0)

    for t in range((NPAD - N) // 16):
        flat = N + t * 16
        u_v[flat >> 7, pl.ds(flat & 127, 16)] = zeros16

    i16 = lax.iota(jnp.int32, 16)
    for j in range(NROWS // 16):
        iota_v[pl.ds(j * 16, 16)] = i16 + j * 16

    @pl.when(s == 0)
    def _():
        pltpu.sync_copy(zero_v, shared)

    plsc.subcore_barrier()

    def deg_step(i, carry):
        r = i >> 3
        co = (i & 7) * 16
        sv = src_v[r, pl.ds(co, 16)]
        plsc.addupdate_scatter(agg_v, [sv >> 7, sv & 127], ones16)
        return carry

    lax.fori_loop(0, ECHUNKS, deg_step, 0)

    pltpu.sync_copy(agg_v, shared.at[iota_v], add=True)
    plsc.subcore_barrier()
    pltpu.sync_copy(shared, dinv_v)
    plsc.subcore_barrier()

    @pl.when(s == 0)
    def _():
        pltpu.sync_copy(zero_v, shared)

    def dinv_row(r, carry):
        for j in range(8):
            sl = pl.ds(j * 16, 16)
            xdeg = dinv_v[r, sl] + 1.0
            bi = 0x5F3759DF - lax.shift_right_logical(plsc.bitcast(xdeg, jnp.int32), 1)
            y = plsc.bitcast(bi, jnp.float32)
            y = y * (1.5 - 0.5 * xdeg * y * y)
            y = y * (1.5 - 0.5 * xdeg * y * y)
            y = y * (1.5 - 0.5 * xdeg * y * y)
            dinv_v[r, sl] = y
            agg_v[r, sl] = zeros16
            w_v[r, sl] = 0.1 * u_v[r, sl]
        return carry

    lax.fori_loop(0, NROWS, dinv_row, 0)
    plsc.subcore_barrier()

    def iteration(k, carry):
        coef = jnp.where(k < NITER, jnp.float32(ALPHA), jnp.float32(1.0))

        def z_row(r, cc):
            for j in range(8):
                sl = pl.ds(j * 16, 16)
                z_v[r, sl] = dinv_v[r, sl] * u_v[r, sl]
            return cc

        lax.fori_loop(0, NROWS, z_row, 0)

        def edge_step(i, cc):
            r = i >> 3
            co = (i & 7) * 16
            sv = src_v[r, pl.ds(co, 16)]
            dv = dst_v[r, pl.ds(co, 16)]
            g = plsc.load_gather(z_v, [sv >> 7, sv & 127])
            plsc.addupdate_scatter(agg_v, [dv >> 7, dv & 127], g)
            return cc

        lax.fori_loop(0, ECHUNKS, edge_step, 0)

        pltpu.sync_copy(agg_v, shared.at[iota_v], add=True)
        plsc.subcore_barrier()
        pltpu.sync_copy(shared, agg_v)
        plsc.subcore_barrier()

        @pl.when(s == 0)
        def _():
            pltpu.sync_copy(zero_v, shared)

        def upd_row(r, cc):
            for j in range(8):
                sl = pl.ds(j * 16, 16)
                un = (1.0 - ALPHA) * dinv_v[r, sl] * (agg_v[r, sl] + z_v[r, sl])
                u_v[r, sl] = un
                w_v[r, sl] = w_v[r, sl] + coef * un
                agg_v[r, sl] = zeros16
            return cc

        lax.fori_loop(0, NROWS, upd_row, 0)
        plsc.subcore_barrier()
        return carry

    lax.fori_loop(1, NITER + 1, iteration, 0)

    @pl.when(jnp.logical_and(c == 0, s == 0))
    def _():
        pltpu.sync_copy(w_v, w_out)


_sc_ppr = pl.kernel(
    _sc_body,
    out_type=jax.ShapeDtypeStruct((NROWS, 128), jnp.float32),
    mesh=plsc.VectorSubcoreMesh(core_axis_name="c", subcore_axis_name="s",
                                num_cores=1),
    compiler_params=pltpu.CompilerParams(needs_layout_passes=False),
    scratch_types=[
        pltpu.VMEM((NROWS, 128), jnp.int32),
        pltpu.VMEM((NROWS, 128), jnp.int32),
        pltpu.VMEM((NROWS, 128), jnp.float32),
        pltpu.VMEM((NROWS, 128), jnp.float32),
        pltpu.VMEM((NROWS, 128), jnp.float32),
        pltpu.VMEM((NROWS, 128), jnp.float32),
        pltpu.VMEM((NROWS, 128), jnp.float32),
        pltpu.VMEM((NROWS, 128), jnp.float32),
        pltpu.VMEM((NROWS,), jnp.int32),
        pltpu.VMEM_SHARED((NROWS, 128), jnp.float32),
    ],
)


def _tc_body(x_ref, w_ref, w0_ref, w1_ref, w2_ref, w3_ref, w4_ref,
             l1w_ref, l1b_ref, l2w_ref, l2b_ref, out_ref, acc_ref):
    i = pl.program_id(0)
    h = jnp.maximum(jnp.dot(x_ref[...], w0_ref[...], preferred_element_type=jnp.float32), 0.0)
    h = jnp.maximum(jnp.dot(h, w1_ref[...], preferred_element_type=jnp.float32), 0.0)
    h = jnp.maximum(jnp.dot(h, w2_ref[...], preferred_element_type=jnp.float32), 0.0)
    h = jnp.maximum(jnp.dot(h, w3_ref[...], preferred_element_type=jnp.float32), 0.0)
    contrib = jnp.dot(w_ref[0], h, preferred_element_type=jnp.float32)

    @pl.when(i == 0)
    def _():
        acc_ref[...] = jnp.zeros_like(acc_ref)

    acc_ref[...] += contrib

    @pl.when(i == NB - 1)
    def _():
        pooled = jnp.dot(acc_ref[...], w4_ref[...], preferred_element_type=jnp.float32)
        h2 = jnp.maximum(
            jnp.dot(pooled, l1w_ref[...], preferred_element_type=jnp.float32) + l1b_ref[...],
            0.0)
        out_ref[...] = jnp.dot(h2, l2w_ref[...], preferred_element_type=jnp.float32) + l2b_ref[...]


def _rep(shape):
    return pl.BlockSpec(shape, lambda i: tuple(0 for _ in shape))


_tc_mlp = pl.pallas_call(
    _tc_body,
    grid=(NB,),
    in_specs=[
        pl.BlockSpec((BN, 256), lambda i: (i, 0)),
        pl.BlockSpec((1, 1, BN), lambda i: (i, 0, 0)),
        _rep((256, 512)),
        _rep((512, 512)),
        _rep((512, 512)),
        _rep((512, 512)),
        _rep((512, 512)),
        _rep((512, 512)),
        _rep((1, 512)),
        _rep((512, 128)),
        _rep((1, 128)),
    ],
    out_specs=pl.BlockSpec((1, 128), lambda i: (0, 0)),
    out_shape=jax.ShapeDtypeStruct((1, 128), jnp.float32),
    scratch_shapes=[pltpu.VMEM((1, 512), jnp.float32)],
)


def kernel(x, edge_index, batch, W0, W1, W2, W3, W4, lin1_W, lin1_b, lin2_W, lin2_b):
    src = edge_index[0].reshape(NTILES, EPT)
    dst = edge_index[1].reshape(NTILES, EPT)
    pad = ((0, 0), (0, EPT_PAD - EPT))
    esrc3 = jnp.pad(src, pad).reshape(NTILES, NROWS, 128)
    edst3 = jnp.pad(dst, pad).reshape(NTILES, NROWS, 128)
    w2d = _sc_ppr(esrc3, edst3)
    w3 = w2d.reshape(NB, 1, BN)

    x_pad = jnp.pad(x, ((0, NPAD - N), (0, 0)))
    out = _tc_mlp(x_pad, w3, W0, W1, W2, W3, W4,
                  lin1_W, lin1_b.reshape(1, 512), lin2_W, lin2_b.reshape(1, 128))
    return out


# speedup vs baseline: 129.2912x; 1.0542x over previous
import functools

import jax
import jax.numpy as jnp
from jax import lax
from jax.experimental import pallas as pl
from jax.experimental.pallas import tpu as pltpu
from jax.experimental.pallas import tpu_sc as plsc

N = 10000
NPAD = 10240
NROWS = 80
E = 160000
NTILES = 16
EPT = E // NTILES
EPT_PAD = NPAD
ECHUNKS = EPT // 16
ALPHA = 0.1
NITER = 10

BN = 1024
NB = NPAD // BN


def _sc_body(esrc, edst, w_out, src_v, dst_v, z_v, u_v, dinv_v, agg_v, w_v,
             zero_v, iota_v, shared):
    c = lax.axis_index("c")
    s = lax.axis_index("s")

    pltpu.sync_copy(esrc.at[s], src_v)
    pltpu.sync_copy(edst.at[s], dst_v)

    ones16 = jnp.ones((16,), jnp.float32)
    zeros16 = jnp.zeros((16,), jnp.float32)

    def init_row(r, carry):
        for j in range(8):
            sl = pl.ds(j * 16, 16)
            zero_v[r, sl] = zeros16
            agg_v[r, sl] = zeros16
            u_v[r, sl] = ones16
        return carry

    lax.fori_loop(0, NROWS, init_row, 0)

    for t in range((NPAD - N) // 16):
        flat = N + t * 16
        u_v[flat >> 7, pl.ds(flat & 127, 16)] = zeros16

    i16 = lax.iota(jnp.int32, 16)
    for j in range(NROWS // 16):
        iota_v[pl.ds(j * 16, 16)] = i16 + j * 16

    @pl.when(s == 0)
    def _():
        pltpu.sync_copy(zero_v, shared)

    plsc.subcore_barrier()

    def deg_step(i, carry):
        r = i >> 3
        co = (i & 7) * 16
        sv = src_v[r, pl.ds(co, 16)]
        plsc.addupdate_scatter(agg_v, [sv >> 7, sv & 127], ones16)
        return carry

    lax.fori_loop(0, ECHUNKS, deg_step, 0)

    pltpu.sync_copy(agg_v, shared.at[iota_v], add=True)
    plsc.subcore_barrier()
    pltpu.sync_copy(shared, dinv_v)
    plsc.subcore_barrier()

    @pl.when(s == 0)
    def _():
        pltpu.sync_copy(zero_v, shared)

    def dinv_row(r, carry):
        for j in range(8):
            sl = pl.ds(j * 16, 16)
            xdeg = dinv_v[r, sl] + 1.0
            bi = 0x5F3759DF - lax.shift_right_logical(plsc.bitcast(xdeg, jnp.int32), 1)
            y = plsc.bitcast(bi, jnp.float32)
            y = y * (1.5 - 0.5 * xdeg * y * y)
            y = y * (1.5 - 0.5 * xdeg * y * y)
            y = y * (1.5 - 0.5 * xdeg * y * y)
            dinv_v[r, sl] = y
            agg_v[r, sl] = zeros16
            u0 = u_v[r, sl]
            w_v[r, sl] = 0.1 * u0
            z_v[r, sl] = y * u0
        return carry

    lax.fori_loop(0, NROWS, dinv_row, 0)
    plsc.subcore_barrier()

    def iteration(k, carry):
        coef = jnp.where(k < NITER, jnp.float32(ALPHA), jnp.float32(1.0))

        def edge_row(r, cc):
            for j in range(8):
                sl = pl.ds(j * 16, 16)
                sv = src_v[r, sl]
                dv = dst_v[r, sl]
                g = plsc.load_gather(z_v, [sv >> 7, sv & 127])
                plsc.addupdate_scatter(agg_v, [dv >> 7, dv & 127], g)
            return cc

        lax.fori_loop(0, EPT // 128, edge_row, 0)
        sv = src_v[EPT // 128, pl.ds(0, 16)]
        dv = dst_v[EPT // 128, pl.ds(0, 16)]
        g = plsc.load_gather(z_v, [sv >> 7, sv & 127])
        plsc.addupdate_scatter(agg_v, [dv >> 7, dv & 127], g)

        pltpu.sync_copy(agg_v, shared.at[iota_v], add=True)
        plsc.subcore_barrier()
        pltpu.sync_copy(shared, agg_v)
        plsc.subcore_barrier()

        @pl.when(s == 0)
        def _():
            pltpu.sync_copy(zero_v, shared)

        def upd_row(r, cc):
            for j in range(8):
                sl = pl.ds(j * 16, 16)
                d16 = dinv_v[r, sl]
                un = (1.0 - ALPHA) * d16 * (agg_v[r, sl] + z_v[r, sl])
                u_v[r, sl] = un
                w_v[r, sl] = w_v[r, sl] + coef * un
                z_v[r, sl] = d16 * un
                agg_v[r, sl] = zeros16
            return cc

        lax.fori_loop(0, NROWS, upd_row, 0)
        plsc.subcore_barrier()
        return carry

    lax.fori_loop(1, NITER + 1, iteration, 0)

    @pl.when(jnp.logical_and(c == 0, s == 0))
    def _():
        pltpu.sync_copy(w_v, w_out)


_sc_ppr = pl.kernel(
    _sc_body,
    out_type=jax.ShapeDtypeStruct((NROWS, 128), jnp.float32),
    mesh=plsc.VectorSubcoreMesh(core_axis_name="c", subcore_axis_name="s",
                                num_cores=1),
    compiler_params=pltpu.CompilerParams(needs_layout_passes=False),
    scratch_types=[
        pltpu.VMEM((NROWS, 128), jnp.int32),
        pltpu.VMEM((NROWS, 128), jnp.int32),
        pltpu.VMEM((NROWS, 128), jnp.float32),
        pltpu.VMEM((NROWS, 128), jnp.float32),
        pltpu.VMEM((NROWS, 128), jnp.float32),
        pltpu.VMEM((NROWS, 128), jnp.float32),
        pltpu.VMEM((NROWS, 128), jnp.float32),
        pltpu.VMEM((NROWS, 128), jnp.float32),
        pltpu.VMEM((NROWS,), jnp.int32),
        pltpu.VMEM_SHARED((NROWS, 128), jnp.float32),
    ],
)


def _tc_body(x_ref, w_ref, w0_ref, w1_ref, w2_ref, w3_ref, w4_ref,
             l1w_ref, l1b_ref, l2w_ref, l2b_ref, out_ref, acc_ref):
    i = pl.program_id(0)
    h = jnp.maximum(jnp.dot(x_ref[...], w0_ref[...], preferred_element_type=jnp.float32), 0.0)
    h = jnp.maximum(jnp.dot(h, w1_ref[...], preferred_element_type=jnp.float32), 0.0)
    h = jnp.maximum(jnp.dot(h, w2_ref[...], preferred_element_type=jnp.float32), 0.0)
    h = jnp.maximum(jnp.dot(h, w3_ref[...], preferred_element_type=jnp.float32), 0.0)
    contrib = jnp.dot(w_ref[0], h, preferred_element_type=jnp.float32)

    @pl.when(i == 0)
    def _():
        acc_ref[...] = jnp.zeros_like(acc_ref)

    acc_ref[...] += contrib

    @pl.when(i == NB - 1)
    def _():
        pooled = jnp.dot(acc_ref[...], w4_ref[...], preferred_element_type=jnp.float32)
        h2 = jnp.maximum(
            jnp.dot(pooled, l1w_ref[...], preferred_element_type=jnp.float32) + l1b_ref[...],
            0.0)
        out_ref[...] = jnp.dot(h2, l2w_ref[...], preferred_element_type=jnp.float32) + l2b_ref[...]


def _rep(shape):
    return pl.BlockSpec(shape, lambda i: tuple(0 for _ in shape))


_tc_mlp = pl.pallas_call(
    _tc_body,
    grid=(NB,),
    in_specs=[
        pl.BlockSpec((BN, 256), lambda i: (i, 0)),
        pl.BlockSpec((1, 1, BN), lambda i: (i, 0, 0)),
        _rep((256, 512)),
        _rep((512, 512)),
        _rep((512, 512)),
        _rep((512, 512)),
        _rep((512, 512)),
        _rep((512, 512)),
        _rep((1, 512)),
        _rep((512, 128)),
        _rep((1, 128)),
    ],
    out_specs=pl.BlockSpec((1, 128), lambda i: (0, 0)),
    out_shape=jax.ShapeDtypeStruct((1, 128), jnp.float32),
    scratch_shapes=[pltpu.VMEM((1, 512), jnp.float32)],
)


def kernel(x, edge_index, batch, W0, W1, W2, W3, W4, lin1_W, lin1_b, lin2_W, lin2_b):
    src = edge_index[0].reshape(NTILES, EPT)
    dst = edge_index[1].reshape(NTILES, EPT)
    pad = ((0, 0), (0, EPT_PAD - EPT))
    esrc3 = jnp.pad(src, pad).reshape(NTILES, NROWS, 128)
    edst3 = jnp.pad(dst, pad).reshape(NTILES, NROWS, 128)
    w2d = _sc_ppr(esrc3, edst3)
    w3 = w2d.reshape(NB, 1, BN)

    x_pad = jnp.pad(x, ((0, NPAD - N), (0, 0)))
    out = _tc_mlp(x_pad, w3, W0, W1, W2, W3, W4,
                  lin1_W, lin1_b.reshape(1, 512), lin2_W, lin2_b.reshape(1, 128))
    return out


# speedup vs baseline: 177.9501x; 1.3764x over previous
import functools

import jax
import jax.numpy as jnp
from jax import lax
from jax.experimental import pallas as pl
from jax.experimental.pallas import tpu as pltpu
from jax.experimental.pallas import tpu_sc as plsc

N = 10000
NPAD = 10240
NROWS = 80
E = 160000
NTILES = 16
EPT = E // NTILES
EPT_PAD = NPAD
ECHUNKS = EPT // 16
ALPHA = 0.1
NITER = 10

BN = 1024
NB = NPAD // BN


def _sc_body(esrc, edst, w_out, src_v, dst_v, z_v, u_v, dinv_v, agg_v, w_v,
             zero_v, iota_v, shared):
    c = lax.axis_index("c")
    s = lax.axis_index("s")

    pltpu.sync_copy(esrc.at[s], src_v)
    pltpu.sync_copy(edst.at[s], dst_v)

    ones16 = jnp.ones((16,), jnp.float32)
    zeros16 = jnp.zeros((16,), jnp.float32)

    def init_row(r, carry):
        for j in range(8):
            sl = pl.ds(j * 16, 16)
            zero_v[r, sl] = zeros16
            agg_v[r, sl] = zeros16
            u_v[r, sl] = ones16
        return carry

    lax.fori_loop(0, NROWS, init_row, 0)

    for t in range((NPAD - N) // 16):
        flat = N + t * 16
        u_v[flat >> 7, pl.ds(flat & 127, 16)] = zeros16

    i16 = lax.iota(jnp.int32, 16)
    for j in range(NROWS // 16):
        iota_v[pl.ds(j * 16, 16)] = i16 + j * 16

    @pl.when(s == 0)
    def _():
        pltpu.sync_copy(zero_v, shared)

    plsc.subcore_barrier()

    def deg_row(r, carry):
        svs = [src_v[r, pl.ds(j * 16, 16)] for j in range(8)]
        for sv in svs:
            plsc.addupdate_scatter(agg_v, [sv >> 7, sv & 127], ones16)
        return carry

    lax.fori_loop(0, EPT // 128, deg_row, 0)
    sv = src_v[EPT // 128, pl.ds(0, 16)]
    plsc.addupdate_scatter(agg_v, [sv >> 7, sv & 127], ones16)

    pltpu.sync_copy(agg_v, shared.at[iota_v], add=True)
    plsc.subcore_barrier()
    pltpu.sync_copy(shared, dinv_v)
    plsc.subcore_barrier()

    @pl.when(s == 0)
    def _():
        pltpu.sync_copy(zero_v, shared)

    def dinv_row(r, carry):
        for j in range(8):
            sl = pl.ds(j * 16, 16)
            xdeg = dinv_v[r, sl] + 1.0
            bi = 0x5F3759DF - lax.shift_right_logical(plsc.bitcast(xdeg, jnp.int32), 1)
            y = plsc.bitcast(bi, jnp.float32)
            y = y * (1.5 - 0.5 * xdeg * y * y)
            y = y * (1.5 - 0.5 * xdeg * y * y)
            y = y * (1.5 - 0.5 * xdeg * y * y)
            dinv_v[r, sl] = y
            agg_v[r, sl] = zeros16
            u0 = u_v[r, sl]
            w_v[r, sl] = 0.1 * u0
            z_v[r, sl] = y * u0
        return carry

    lax.fori_loop(0, NROWS, dinv_row, 0)
    plsc.subcore_barrier()

    def iteration(k, carry):
        coef = jnp.where(k < NITER, jnp.float32(ALPHA), jnp.float32(1.0))

        def edge_row(r, cc):
            svs = [src_v[r, pl.ds(j * 16, 16)] for j in range(8)]
            dvs = [dst_v[r, pl.ds(j * 16, 16)] for j in range(8)]
            gs = [plsc.load_gather(z_v, [sv >> 7, sv & 127]) for sv in svs]
            for dv, g in zip(dvs, gs):
                plsc.addupdate_scatter(agg_v, [dv >> 7, dv & 127], g)
            return cc

        lax.fori_loop(0, EPT // 128, edge_row, 0)
        sv = src_v[EPT // 128, pl.ds(0, 16)]
        dv = dst_v[EPT // 128, pl.ds(0, 16)]
        g = plsc.load_gather(z_v, [sv >> 7, sv & 127])
        plsc.addupdate_scatter(agg_v, [dv >> 7, dv & 127], g)

        pltpu.sync_copy(agg_v, shared.at[iota_v], add=True)
        plsc.subcore_barrier()
        pltpu.sync_copy(shared, agg_v)
        plsc.subcore_barrier()

        @pl.when(s == 0)
        def _():
            pltpu.sync_copy(zero_v, shared)

        def upd_row(r, cc):
            for j in range(8):
                sl = pl.ds(j * 16, 16)
                d16 = dinv_v[r, sl]
                un = (1.0 - ALPHA) * d16 * (agg_v[r, sl] + z_v[r, sl])
                u_v[r, sl] = un
                w_v[r, sl] = w_v[r, sl] + coef * un
                z_v[r, sl] = d16 * un
                agg_v[r, sl] = zeros16
            return cc

        lax.fori_loop(0, NROWS, upd_row, 0)
        plsc.subcore_barrier()
        return carry

    lax.fori_loop(1, NITER + 1, iteration, 0)

    @pl.when(jnp.logical_and(c == 0, s == 0))
    def _():
        pltpu.sync_copy(w_v, w_out)


_sc_ppr = pl.kernel(
    _sc_body,
    out_type=jax.ShapeDtypeStruct((NROWS, 128), jnp.float32),
    mesh=plsc.VectorSubcoreMesh(core_axis_name="c", subcore_axis_name="s",
                                num_cores=1),
    compiler_params=pltpu.CompilerParams(needs_layout_passes=False),
    scratch_types=[
        pltpu.VMEM((NROWS, 128), jnp.int32),
        pltpu.VMEM((NROWS, 128), jnp.int32),
        pltpu.VMEM((NROWS, 128), jnp.float32),
        pltpu.VMEM((NROWS, 128), jnp.float32),
        pltpu.VMEM((NROWS, 128), jnp.float32),
        pltpu.VMEM((NROWS, 128), jnp.float32),
        pltpu.VMEM((NROWS, 128), jnp.float32),
        pltpu.VMEM((NROWS, 128), jnp.float32),
        pltpu.VMEM((NROWS,), jnp.int32),
        pltpu.VMEM_SHARED((NROWS, 128), jnp.float32),
    ],
)


def _tc_body(x_ref, w_ref, w0_ref, w1_ref, w2_ref, w3_ref, w4_ref,
             l1w_ref, l1b_ref, l2w_ref, l2b_ref, out_ref, acc_ref):
    i = pl.program_id(0)
    h = jnp.maximum(jnp.dot(x_ref[...], w0_ref[...], preferred_element_type=jnp.float32), 0.0)
    h = jnp.maximum(jnp.dot(h, w1_ref[...], preferred_element_type=jnp.float32), 0.0)
    h = jnp.maximum(jnp.dot(h, w2_ref[...], preferred_element_type=jnp.float32), 0.0)
    h = jnp.maximum(jnp.dot(h, w3_ref[...], preferred_element_type=jnp.float32), 0.0)
    contrib = jnp.dot(w_ref[0], h, preferred_element_type=jnp.float32)

    @pl.when(i == 0)
    def _():
        acc_ref[...] = jnp.zeros_like(acc_ref)

    acc_ref[...] += contrib

    @pl.when(i == NB - 1)
    def _():
        pooled = jnp.dot(acc_ref[...], w4_ref[...], preferred_element_type=jnp.float32)
        h2 = jnp.maximum(
            jnp.dot(pooled, l1w_ref[...], preferred_element_type=jnp.float32) + l1b_ref[...],
            0.0)
        out_ref[...] = jnp.dot(h2, l2w_ref[...], preferred_element_type=jnp.float32) + l2b_ref[...]


def _rep(shape):
    return pl.BlockSpec(shape, lambda i: tuple(0 for _ in shape))


_tc_mlp = pl.pallas_call(
    _tc_body,
    grid=(NB,),
    in_specs=[
        pl.BlockSpec((BN, 256), lambda i: (i, 0)),
        pl.BlockSpec((1, 1, BN), lambda i: (i, 0, 0)),
        _rep((256, 512)),
        _rep((512, 512)),
        _rep((512, 512)),
        _rep((512, 512)),
        _rep((512, 512)),
        _rep((512, 512)),
        _rep((1, 512)),
        _rep((512, 128)),
        _rep((1, 128)),
    ],
    out_specs=pl.BlockSpec((1, 128), lambda i: (0, 0)),
    out_shape=jax.ShapeDtypeStruct((1, 128), jnp.float32),
    scratch_shapes=[pltpu.VMEM((1, 512), jnp.float32)],
)


def kernel(x, edge_index, batch, W0, W1, W2, W3, W4, lin1_W, lin1_b, lin2_W, lin2_b):
    src = edge_index[0].reshape(NTILES, EPT)
    dst = edge_index[1].reshape(NTILES, EPT)
    pad = ((0, 0), (0, EPT_PAD - EPT))
    esrc3 = jnp.pad(src, pad).reshape(NTILES, NROWS, 128)
    edst3 = jnp.pad(dst, pad).reshape(NTILES, NROWS, 128)
    w2d = _sc_ppr(esrc3, edst3)
    w3 = w2d.reshape(NB, 1, BN)

    x_pad = jnp.pad(x, ((0, NPAD - N), (0, 0)))
    out = _tc_mlp(x_pad, w3, W0, W1, W2, W3, W4,
                  lin1_W, lin1_b.reshape(1, 512), lin2_W, lin2_b.reshape(1, 128))
    return out


# speedup vs baseline: 205.5826x; 1.1553x over previous
import functools

import jax
import jax.numpy as jnp
from jax import lax
from jax.experimental import pallas as pl
from jax.experimental.pallas import tpu as pltpu
from jax.experimental.pallas import tpu_sc as plsc

N = 10000
NPAD = 10240
NROWS = 80
E = 160000
NTILES = 16
EPT = E // NTILES
EPT_PAD = NPAD
ECHUNKS = EPT // 16
ALPHA = 0.1
NITER = 10

BN = 1024
NB = NPAD // BN


def _sc_body(esrc, edst, w_out, src_v, dst_v, z_v, u_v, dinv_v, agg_v, w_v,
             zero_v, iota_v, shared):
    c = lax.axis_index("c")
    s = lax.axis_index("s")

    pltpu.sync_copy(esrc.at[s], src_v)
    pltpu.sync_copy(edst.at[s], dst_v)

    ones16 = jnp.ones((16,), jnp.float32)
    zeros16 = jnp.zeros((16,), jnp.float32)

    def init_row(r, carry):
        for j in range(8):
            sl = pl.ds(j * 16, 16)
            zero_v[r, sl] = zeros16
            agg_v[r, sl] = zeros16
            u_v[r, sl] = ones16
        return carry

    lax.fori_loop(0, NROWS, init_row, 0)

    for t in range((NPAD - N) // 16):
        flat = N + t * 16
        u_v[flat >> 7, pl.ds(flat & 127, 16)] = zeros16

    i16 = lax.iota(jnp.int32, 16)
    for j in range(NROWS // 16):
        iota_v[pl.ds(j * 16, 16)] = i16 + j * 16

    @pl.when(s == 0)
    def _():
        pltpu.sync_copy(zero_v, shared)

    plsc.subcore_barrier()

    def deg_row(r, carry):
        svs = [src_v[r, pl.ds(j * 16, 16)] for j in range(8)]
        for sv in svs:
            plsc.addupdate_scatter(agg_v, [sv >> 7, sv & 127], ones16)
        return carry

    lax.fori_loop(0, EPT // 128, deg_row, 0)
    sv = src_v[EPT // 128, pl.ds(0, 16)]
    plsc.addupdate_scatter(agg_v, [sv >> 7, sv & 127], ones16)

    pltpu.sync_copy(agg_v, shared.at[iota_v], add=True)
    plsc.subcore_barrier()
    pltpu.sync_copy(shared, dinv_v)
    plsc.subcore_barrier()

    @pl.when(s == 0)
    def _():
        pltpu.sync_copy(zero_v, shared)

    def dinv_row(r, carry):
        for j in range(8):
            sl = pl.ds(j * 16, 16)
            xdeg = dinv_v[r, sl] + 1.0
            bi = 0x5F3759DF - lax.shift_right_logical(plsc.bitcast(xdeg, jnp.int32), 1)
            y = plsc.bitcast(bi, jnp.float32)
            y = y * (1.5 - 0.5 * xdeg * y * y)
            y = y * (1.5 - 0.5 * xdeg * y * y)
            y = y * (1.5 - 0.5 * xdeg * y * y)
            dinv_v[r, sl] = y
            agg_v[r, sl] = zeros16
            u0 = u_v[r, sl]
            w_v[r, sl] = 0.1 * u0
            z_v[r, sl] = y * u0
        return carry

    lax.fori_loop(0, NROWS, dinv_row, 0)
    plsc.subcore_barrier()

    def iteration(k, carry):
        coef = jnp.where(k < NITER, jnp.float32(ALPHA), jnp.float32(1.0))

        def edge_row(r, cc):
            svs = [src_v[r, pl.ds(j * 16, 16)] for j in range(8)]
            dvs = [dst_v[r, pl.ds(j * 16, 16)] for j in range(8)]
            gs = [plsc.load_gather(z_v, [sv >> 7, sv & 127]) for sv in svs]
            for dv, g in zip(dvs, gs):
                plsc.addupdate_scatter(agg_v, [dv >> 7, dv & 127], g)
            return cc

        lax.fori_loop(0, EPT // 128, edge_row, 0)
        sv = src_v[EPT // 128, pl.ds(0, 16)]
        dv = dst_v[EPT // 128, pl.ds(0, 16)]
        g = plsc.load_gather(z_v, [sv >> 7, sv & 127])
        plsc.addupdate_scatter(agg_v, [dv >> 7, dv & 127], g)

        pltpu.sync_copy(agg_v, shared.at[iota_v], add=True)
        plsc.subcore_barrier()
        pltpu.sync_copy(shared, agg_v)
        plsc.subcore_barrier()

        @pl.when(s == 0)
        def _():
            pltpu.sync_copy(zero_v, shared)

        def upd_row(r, cc):
            for j in range(8):
                sl = pl.ds(j * 16, 16)
                d16 = dinv_v[r, sl]
                un = (1.0 - ALPHA) * d16 * (agg_v[r, sl] + z_v[r, sl])
                u_v[r, sl] = un
                w_v[r, sl] = w_v[r, sl] + coef * un
                z_v[r, sl] = d16 * un
                agg_v[r, sl] = zeros16
            return cc

        lax.fori_loop(0, NROWS, upd_row, 0)
        plsc.subcore_barrier()
        return carry

    lax.fori_loop(1, NITER + 1, iteration, 0)

    @pl.when(jnp.logical_and(c == 0, s == 0))
    def _():
        pltpu.sync_copy(w_v, w_out)


_sc_ppr = pl.kernel(
    _sc_body,
    out_type=jax.ShapeDtypeStruct((NROWS, 128), jnp.float32),
    mesh=plsc.VectorSubcoreMesh(core_axis_name="c", subcore_axis_name="s",
                                num_cores=1),
    compiler_params=pltpu.CompilerParams(needs_layout_passes=False),
    scratch_types=[
        pltpu.VMEM((NROWS, 128), jnp.int32),
        pltpu.VMEM((NROWS, 128), jnp.int32),
        pltpu.VMEM((NROWS, 128), jnp.float32),
        pltpu.VMEM((NROWS, 128), jnp.float32),
        pltpu.VMEM((NROWS, 128), jnp.float32),
        pltpu.VMEM((NROWS, 128), jnp.float32),
        pltpu.VMEM((NROWS, 128), jnp.float32),
        pltpu.VMEM((NROWS, 128), jnp.float32),
        pltpu.VMEM((NROWS,), jnp.int32),
        pltpu.VMEM_SHARED((NROWS, 128), jnp.float32),
    ],
)


def _rep(shape):
    return pl.BlockSpec(shape, lambda i: tuple(0 for _ in shape))


def _tc_hidden_body(x_ref, w0_ref, w1_ref, w2_ref, w3_ref, h_ref):
    h = jnp.maximum(jnp.dot(x_ref[...], w0_ref[...], preferred_element_type=jnp.float32), 0.0)
    h = jnp.maximum(jnp.dot(h, w1_ref[...], preferred_element_type=jnp.float32), 0.0)
    h = jnp.maximum(jnp.dot(h, w2_ref[...], preferred_element_type=jnp.float32), 0.0)
    h_ref[...] = jnp.maximum(jnp.dot(h, w3_ref[...], preferred_element_type=jnp.float32), 0.0)


_tc_hidden = pl.pallas_call(
    _tc_hidden_body,
    grid=(NB,),
    in_specs=[
        pl.BlockSpec((BN, 256), lambda i: (i, 0)),
        _rep((256, 512)),
        _rep((512, 512)),
        _rep((512, 512)),
        _rep((512, 512)),
    ],
    out_specs=pl.BlockSpec((BN, 512), lambda i: (i, 0)),
    out_shape=jax.ShapeDtypeStruct((NPAD, 512), jnp.float32),
)


def _tc_comb_body(w_ref, h_ref, w4_ref, l1w_ref, l1b_ref, l2w_ref, l2b_ref,
                  out_ref, acc_ref):
    i = pl.program_id(0)
    contrib = jnp.dot(w_ref[0], h_ref[...], preferred_element_type=jnp.float32)

    @pl.when(i == 0)
    def _():
        acc_ref[...] = jnp.zeros_like(acc_ref)

    acc_ref[...] += contrib

    @pl.when(i == NB - 1)
    def _():
        pooled = jnp.dot(acc_ref[...], w4_ref[...], preferred_element_type=jnp.float32)
        h2 = jnp.maximum(
            jnp.dot(pooled, l1w_ref[...], preferred_element_type=jnp.float32) + l1b_ref[...],
            0.0)
        out_ref[...] = jnp.dot(h2, l2w_ref[...], preferred_element_type=jnp.float32) + l2b_ref[...]


_tc_combine = pl.pallas_call(
    _tc_comb_body,
    grid=(NB,),
    in_specs=[
        pl.BlockSpec((1, 1, BN), lambda i: (i, 0, 0)),
        pl.BlockSpec((BN, 512), lambda i: (i, 0)),
        _rep((512, 512)),
        _rep((512, 512)),
        _rep((1, 512)),
        _rep((512, 128)),
        _rep((1, 128)),
    ],
    out_specs=pl.BlockSpec((1, 128), lambda i: (0, 0)),
    out_shape=jax.ShapeDtypeStruct((1, 128), jnp.float32),
    scratch_shapes=[pltpu.VMEM((1, 512), jnp.float32)],
)


def kernel(x, edge_index, batch, W0, W1, W2, W3, W4, lin1_W, lin1_b, lin2_W, lin2_b):
    src = edge_index[0].reshape(NTILES, EPT)
    dst = edge_index[1].reshape(NTILES, EPT)
    pad = ((0, 0), (0, EPT_PAD - EPT))
    esrc3 = jnp.pad(src, pad).reshape(NTILES, NROWS, 128)
    edst3 = jnp.pad(dst, pad).reshape(NTILES, NROWS, 128)
    w2d = _sc_ppr(esrc3, edst3)
    w3 = w2d.reshape(NB, 1, BN)

    x_pad = jnp.pad(x, ((0, NPAD - N), (0, 0)))
    hidden = _tc_hidden(x_pad, W0, W1, W2, W3)
    out = _tc_combine(w3, hidden, W4,
                      lin1_W, lin1_b.reshape(1, 512), lin2_W, lin2_b.reshape(1, 128))
    return out
